# baseline (speedup 1.0000x reference)
import functools

import jax
import jax.numpy as jnp
from jax import lax
from jax.experimental import pallas as pl
from jax.experimental.pallas import tpu as pltpu
from jax.experimental.pallas import tpu_sc as plsc

K = 8192
D = 256
N = 4096
B = 4
RB = N // B
CB = 4096


def _argmin_body(x_ref, c_ref, idx_ref, mn_ref, mi_ref, xs_ref):
    j = pl.program_id(1)
    nj = pl.num_programs(1)

    @pl.when(j == 0)
    def _():
        mn_ref[...] = jnp.full((RB, 1), jnp.inf, jnp.float32)
        mi_ref[...] = jnp.zeros((RB, 1), jnp.float32)
        xs_ref[...] = lax.transpose(x_ref[...].reshape(D, RB), (1, 0))

    x = xs_ref[...]
    c = c_ref[...]
    xn = jnp.sum(x * x, axis=1, keepdims=True)
    cn = jnp.sum(c * c, axis=1)[None, :]
    mm2 = lax.dot_general(x * (-2.0), c, (((1,), (1,)), ((), ())),
                          preferred_element_type=jnp.float32)
    d = (xn + cn) + mm2
    m_loc = jnp.min(d, axis=1, keepdims=True)
    cols = lax.broadcasted_iota(jnp.int32, (1, CB), 1).astype(jnp.float32)
    i_loc = jnp.min(jnp.where(d == m_loc, cols, jnp.inf), axis=1, keepdims=True)
    better = m_loc < mn_ref[...]
    mi_ref[...] = jnp.where(better, i_loc + (j * CB).astype(jnp.float32),
                            mi_ref[...])
    mn_ref[...] = jnp.where(better, m_loc, mn_ref[...])

    @pl.when(j == nj - 1)
    def _():
        idx_ref[...] = mi_ref[...].astype(jnp.int32)


def _argmin_indices(x3, codebook):
    return pl.pallas_call(
        _argmin_body,
        grid=(B, K // CB),
        in_specs=[
            pl.BlockSpec((1, D, RB), lambda i, j: (i, 0, 0)),
            pl.BlockSpec((CB, D), lambda i, j: (j, 0)),
        ],
        out_specs=pl.BlockSpec((RB, 1), lambda i, j: (i, 0)),
        out_shape=jax.ShapeDtypeStruct((N, 1), jnp.int32),
        scratch_shapes=[
            pltpu.VMEM((RB, 1), jnp.float32),
            pltpu.VMEM((RB, 1), jnp.float32),
            pltpu.VMEM((RB, D), jnp.float32),
        ],
    )(x3, codebook)


def _make_sc_gather():
    info = plsc.get_sparse_core_info()
    nw = info.num_cores * info.num_subcores
    bpw = N // nw
    mesh = plsc.VectorSubcoreMesh(core_axis_name="c", subcore_axis_name="s")

    @functools.partial(
        pl.kernel,
        mesh=mesh,
        out_type=jax.ShapeDtypeStruct((N, D), jnp.float32),
        scratch_types=[
            pltpu.VMEM((bpw,), jnp.int32),
            pltpu.VMEM((bpw, D), jnp.float32),
            pltpu.SemaphoreType.DMA,
        ],
    )
    def gather_k(idx_hbm, table_hbm, out_hbm, idx_v, rows_v, sem):
        wid = lax.axis_index("s") * info.num_cores + lax.axis_index("c")
        base = wid * bpw
        pltpu.sync_copy(idx_hbm.at[pl.ds(base, bpw)], idx_v)
        pltpu.async_copy(table_hbm.at[idx_v], rows_v, sem).wait()
        pltpu.sync_copy(rows_v, out_hbm.at[pl.ds(base, bpw)])

    return gather_k


_sc_gather_cache = []


def _sc_gather(idx, table):
    if not _sc_gather_cache:
        _sc_gather_cache.append(_make_sc_gather())
    return _sc_gather_cache[0](idx, table)


def _finalize_body(x_ref, q_ref, quant_ref, loss_ref, acc_ref):
    b = pl.program_id(0)
    nb = pl.num_programs(0)
    x = x_ref[...].reshape(D, RB)
    qt = lax.transpose(q_ref[...], (1, 0))
    dqx = qt - x
    quant_ref[...] = (x + dqx).reshape(1, D, RB)
    s = jnp.sum(dqx * dqx)

    @pl.when(b == 0)
    def _():
        acc_ref[0] = 0.0

    acc_ref[0] += s

    @pl.when(b == nb - 1)
    def _():
        loss_ref[...] = (1.25 * (acc_ref[0] * (1.0 / (N * D)))).reshape(1, 1)


def _finalize(x3, q):
    return pl.pallas_call(
        _finalize_body,
        grid=(B,),
        in_specs=[
            pl.BlockSpec((1, D, RB), lambda b: (b, 0, 0)),
            pl.BlockSpec((RB, D), lambda b: (b, 0)),
        ],
        out_specs=[
            pl.BlockSpec((1, D, RB), lambda b: (b, 0, 0)),
            pl.BlockSpec((1, 1), lambda b: (0, 0)),
        ],
        out_shape=[
            jax.ShapeDtypeStruct((B, D, RB), jnp.float32),
            jax.ShapeDtypeStruct((1, 1), jnp.float32),
        ],
        scratch_shapes=[pltpu.SMEM((1,), jnp.float32)],
    )(x3, q)


def kernel(inputs, codebook):
    x3 = inputs.reshape(B, D, RB)
    idx = _argmin_indices(x3, codebook).reshape(N)
    q = _sc_gather(idx, codebook)
    quant3, loss = _finalize(x3, q)
    quant = quant3.reshape(inputs.shape)
    return (quant, loss.reshape(()), idx)


# speedup vs baseline: 1.0658x; 1.0658x over previous
import functools

import jax
import jax.numpy as jnp
from jax import lax
from jax.experimental import pallas as pl
from jax.experimental.pallas import tpu as pltpu
from jax.experimental.pallas import tpu_sc as plsc

K = 8192
D = 256
N = 4096
B = 4
RB = N // B
CB = 4096


def _argmin_body(x_ref, c_ref, idx_ref, mn_ref, mi_ref):
    j = pl.program_id(1)
    nj = pl.num_programs(1)

    @pl.when(j == 0)
    def _():
        mn_ref[...] = jnp.full((RB, 1), jnp.inf, jnp.float32)
        mi_ref[...] = jnp.zeros((RB, 1), jnp.float32)

    x = x_ref[...]
    c = c_ref[...]
    xn = jnp.sum(x * x, axis=1, keepdims=True)
    cn = jnp.sum(c * c, axis=1)[None, :]
    mm2 = lax.dot_general(x * (-2.0), c, (((1,), (1,)), ((), ())),
                          preferred_element_type=jnp.float32)
    d = (xn + cn) + mm2
    m_loc = jnp.min(d, axis=1, keepdims=True)
    cols = lax.broadcasted_iota(jnp.int32, (1, CB), 1).astype(jnp.float32)
    i_loc = jnp.min(jnp.where(d == m_loc, cols, jnp.inf), axis=1, keepdims=True)
    better = m_loc < mn_ref[...]
    mi_ref[...] = jnp.where(better, i_loc + (j * CB).astype(jnp.float32),
                            mi_ref[...])
    mn_ref[...] = jnp.where(better, m_loc, mn_ref[...])

    @pl.when(j == nj - 1)
    def _():
        idx_ref[...] = mi_ref[...].astype(jnp.int32)


def _argmin_indices(flat, codebook):
    return pl.pallas_call(
        _argmin_body,
        grid=(B, K // CB),
        in_specs=[
            pl.BlockSpec((RB, D), lambda i, j: (i, 0)),
            pl.BlockSpec((CB, D), lambda i, j: (j, 0)),
        ],
        out_specs=pl.BlockSpec((RB, 1), lambda i, j: (i, 0)),
        out_shape=jax.ShapeDtypeStruct((N, 1), jnp.int32),
        scratch_shapes=[
            pltpu.VMEM((RB, 1), jnp.float32),
            pltpu.VMEM((RB, 1), jnp.float32),
        ],
    )(flat, codebook)


def _make_sc_gather():
    info = plsc.get_sparse_core_info()
    nw = info.num_cores * info.num_subcores
    bpw = N // nw
    mesh = plsc.VectorSubcoreMesh(core_axis_name="c", subcore_axis_name="s")

    @functools.partial(
        pl.kernel,
        mesh=mesh,
        out_type=jax.ShapeDtypeStruct((N, D), jnp.float32),
        scratch_types=[
            pltpu.VMEM((bpw,), jnp.int32),
            pltpu.VMEM((bpw, D), jnp.float32),
            pltpu.SemaphoreType.DMA,
        ],
    )
    def gather_k(idx_hbm, table_hbm, out_hbm, idx_v, rows_v, sem):
        wid = lax.axis_index("s") * info.num_cores + lax.axis_index("c")
        base = wid * bpw
        pltpu.sync_copy(idx_hbm.at[pl.ds(base, bpw)], idx_v)
        pltpu.async_copy(table_hbm.at[idx_v], rows_v, sem).wait()
        pltpu.sync_copy(rows_v, out_hbm.at[pl.ds(base, bpw)])

    return gather_k


_sc_gather_cache = []


def _sc_gather(idx, table):
    if not _sc_gather_cache:
        _sc_gather_cache.append(_make_sc_gather())
    return _sc_gather_cache[0](idx, table)


def _finalize_body(x_ref, q_ref, quant_ref, loss_ref, acc_ref):
    b = pl.program_id(0)
    nb = pl.num_programs(0)
    x = x_ref[...].reshape(D, RB)
    qt = lax.transpose(q_ref[...], (1, 0))
    dqx = qt - x
    quant_ref[...] = (x + dqx).reshape(1, D, RB)
    s = jnp.sum(dqx * dqx)

    @pl.when(b == 0)
    def _():
        acc_ref[0] = 0.0

    acc_ref[0] += s

    @pl.when(b == nb - 1)
    def _():
        loss_ref[...] = (1.25 * (acc_ref[0] * (1.0 / (N * D)))).reshape(1, 1)


def _finalize(x3, q):
    return pl.pallas_call(
        _finalize_body,
        grid=(B,),
        in_specs=[
            pl.BlockSpec((1, D, RB), lambda b: (b, 0, 0)),
            pl.BlockSpec((RB, D), lambda b: (b, 0)),
        ],
        out_specs=[
            pl.BlockSpec((1, D, RB), lambda b: (b, 0, 0)),
            pl.BlockSpec((1, 1), lambda b: (0, 0)),
        ],
        out_shape=[
            jax.ShapeDtypeStruct((B, D, RB), jnp.float32),
            jax.ShapeDtypeStruct((1, 1), jnp.float32),
        ],
        scratch_shapes=[pltpu.SMEM((1,), jnp.float32)],
    )(x3, q)


def kernel(inputs, codebook):
    x3 = inputs.reshape(B, D, RB)
    flat = jnp.transpose(inputs, (0, 2, 3, 1)).reshape(-1, D)
    idx = _argmin_indices(flat, codebook).reshape(N)
    q = _sc_gather(idx, codebook)
    quant3, loss = _finalize(x3, q)
    quant = quant3.reshape(inputs.shape)
    return (quant, loss.reshape(()), idx)


# speedup vs baseline: 1.0892x; 1.0219x over previous
import functools

import jax
import jax.numpy as jnp
from jax import lax
from jax.experimental import pallas as pl
from jax.experimental.pallas import tpu as pltpu
from jax.experimental.pallas import tpu_sc as plsc

K = 8192
D = 256
N = 4096
B = 4
RB = N // B
CB = 4096


def _argmin_body(x_ref, c_ref, idx_ref, mn_ref, mi_ref):
    j = pl.program_id(1)
    nj = pl.num_programs(1)

    @pl.when(j == 0)
    def _():
        mn_ref[...] = jnp.full((RB, 1), jnp.inf, jnp.float32)
        mi_ref[...] = jnp.zeros((RB, 1), jnp.float32)

    x = x_ref[...]
    c = c_ref[...]
    xn = jnp.sum(x * x, axis=1, keepdims=True)
    cn = jnp.sum(c * c, axis=1)[None, :]
    mm2 = lax.dot_general(x * (-2.0), c, (((1,), (1,)), ((), ())),
                          preferred_element_type=jnp.float32)
    d = (xn + cn) + mm2
    m_loc = jnp.min(d, axis=1, keepdims=True)
    cols = lax.broadcasted_iota(jnp.int32, (1, CB), 1).astype(jnp.float32)
    i_loc = jnp.min(jnp.where(d == m_loc, cols, jnp.inf), axis=1, keepdims=True)
    better = m_loc < mn_ref[...]
    mi_ref[...] = jnp.where(better, i_loc + (j * CB).astype(jnp.float32),
                            mi_ref[...])
    mn_ref[...] = jnp.where(better, m_loc, mn_ref[...])

    @pl.when(j == nj - 1)
    def _():
        idx_ref[...] = mi_ref[...].astype(jnp.int32)


def _argmin_indices(flat, codebook):
    return pl.pallas_call(
        _argmin_body,
        grid=(B, K // CB),
        in_specs=[
            pl.BlockSpec((RB, D), lambda i, j: (i, 0)),
            pl.BlockSpec((CB, D), lambda i, j: (j, 0)),
        ],
        out_specs=pl.BlockSpec((RB, 1), lambda i, j: (i, 0)),
        out_shape=jax.ShapeDtypeStruct((N, 1), jnp.int32),
        scratch_shapes=[
            pltpu.VMEM((RB, 1), jnp.float32),
            pltpu.VMEM((RB, 1), jnp.float32),
        ],
    )(flat, codebook)


def _make_sc_gather():
    info = plsc.get_sparse_core_info()
    nw = info.num_cores * info.num_subcores
    bpw = N // nw
    mesh = plsc.VectorSubcoreMesh(core_axis_name="c", subcore_axis_name="s")

    @functools.partial(
        pl.kernel,
        mesh=mesh,
        out_type=jax.ShapeDtypeStruct((N, D), jnp.float32),
        scratch_types=[
            pltpu.VMEM((bpw,), jnp.int32),
            pltpu.VMEM((bpw, D), jnp.float32),
            pltpu.SemaphoreType.DMA,
        ],
    )
    def gather_k(idx_hbm, table_hbm, out_hbm, idx_v, rows_v, sem):
        wid = lax.axis_index("s") * info.num_cores + lax.axis_index("c")
        base = wid * bpw
        pltpu.sync_copy(idx_hbm.at[pl.ds(base, bpw)], idx_v)
        pltpu.async_copy(table_hbm.at[idx_v], rows_v, sem).wait()
        pltpu.sync_copy(rows_v, out_hbm.at[pl.ds(base, bpw)])

    return gather_k


_sc_gather_cache = []


def _sc_gather(idx, table):
    if not _sc_gather_cache:
        _sc_gather_cache.append(_make_sc_gather())
    return _sc_gather_cache[0](idx, table)


def _finalize_body(x_ref, q_ref, quant_ref, loss_ref, acc_ref):
    b = pl.program_id(0)
    nb = pl.num_programs(0)
    x = x_ref[...]
    q = q_ref[...]
    dqx = q - x
    quant_ref[...] = lax.transpose(x + dqx, (1, 0)).reshape(1, D, RB)
    s = jnp.sum(dqx * dqx)

    @pl.when(b == 0)
    def _():
        acc_ref[0] = 0.0

    acc_ref[0] += s

    @pl.when(b == nb - 1)
    def _():
        loss_ref[...] = (1.25 * (acc_ref[0] * (1.0 / (N * D)))).reshape(1, 1)


def _finalize(flat, q):
    return pl.pallas_call(
        _finalize_body,
        grid=(B,),
        in_specs=[
            pl.BlockSpec((RB, D), lambda b: (b, 0)),
            pl.BlockSpec((RB, D), lambda b: (b, 0)),
        ],
        out_specs=[
            pl.BlockSpec((1, D, RB), lambda b: (b, 0, 0)),
            pl.BlockSpec((1, 1), lambda b: (0, 0)),
        ],
        out_shape=[
            jax.ShapeDtypeStruct((B, D, RB), jnp.float32),
            jax.ShapeDtypeStruct((1, 1), jnp.float32),
        ],
        scratch_shapes=[pltpu.SMEM((1,), jnp.float32)],
    )(flat, q)


def kernel(inputs, codebook):
    flat = jnp.transpose(inputs, (0, 2, 3, 1)).reshape(-1, D)
    idx = _argmin_indices(flat, codebook).reshape(N)
    q = _sc_gather(idx, codebook)
    quant3, loss = _finalize(flat, q)
    quant = quant3.reshape(inputs.shape)
    return (quant, loss.reshape(()), idx)


# speedup vs baseline: 1.1739x; 1.0778x over previous
import functools

import jax
import jax.numpy as jnp
from jax import lax
from jax.experimental import pallas as pl
from jax.experimental.pallas import tpu as pltpu
from jax.experimental.pallas import tpu_sc as plsc

K = 8192
D = 256
N = 4096
B = 4
RB = N // B
CB = 4096


def _argmin_body(x_ref, c_ref, idx_ref, mn_ref, mi_ref):
    j = pl.program_id(1)
    nj = pl.num_programs(1)

    @pl.when(j == 0)
    def _():
        mn_ref[...] = jnp.full((RB, 1), jnp.inf, jnp.float32)
        mi_ref[...] = jnp.zeros((RB, 1), jnp.float32)

    x = x_ref[...]
    c = c_ref[...]
    xn = jnp.sum(x * x, axis=1, keepdims=True)
    cn = jnp.sum(c * c, axis=1)[None, :]
    mm2 = lax.dot_general(x * (-2.0), c, (((1,), (1,)), ((), ())),
                          preferred_element_type=jnp.float32)
    d = (xn + cn) + mm2
    m_loc = jnp.min(d, axis=1, keepdims=True)
    cols = lax.broadcasted_iota(jnp.int32, (1, CB), 1).astype(jnp.float32)
    i_loc = jnp.min(jnp.where(d == m_loc, cols, jnp.inf), axis=1, keepdims=True)
    better = m_loc < mn_ref[...]
    mi_ref[...] = jnp.where(better, i_loc + (j * CB).astype(jnp.float32),
                            mi_ref[...])
    mn_ref[...] = jnp.where(better, m_loc, mn_ref[...])

    @pl.when(j == nj - 1)
    def _():
        idx_ref[...] = mi_ref[...].astype(jnp.int32)


def _argmin_indices(flat, codebook):
    return pl.pallas_call(
        _argmin_body,
        grid=(B, K // CB),
        in_specs=[
            pl.BlockSpec((RB, D), lambda i, j: (i, 0)),
            pl.BlockSpec((CB, D), lambda i, j: (j, 0)),
        ],
        out_specs=pl.BlockSpec((RB, 1), lambda i, j: (i, 0)),
        out_shape=jax.ShapeDtypeStruct((N, 1), jnp.int32),
        scratch_shapes=[
            pltpu.VMEM((RB, 1), jnp.float32),
            pltpu.VMEM((RB, 1), jnp.float32),
        ],
        compiler_params=pltpu.CompilerParams(
            dimension_semantics=("parallel", "arbitrary")),
    )(flat, codebook)


def _make_sc_gather():
    info = plsc.get_sparse_core_info()
    nw = info.num_cores * info.num_subcores
    bpw = N // nw
    mesh = plsc.VectorSubcoreMesh(core_axis_name="c", subcore_axis_name="s")

    @functools.partial(
        pl.kernel,
        mesh=mesh,
        out_type=jax.ShapeDtypeStruct((N, D), jnp.float32),
        scratch_types=[
            pltpu.VMEM((bpw,), jnp.int32),
            pltpu.VMEM((bpw, D), jnp.float32),
            pltpu.SemaphoreType.DMA,
        ],
    )
    def gather_k(idx_hbm, table_hbm, out_hbm, idx_v, rows_v, sem):
        wid = lax.axis_index("s") * info.num_cores + lax.axis_index("c")
        base = wid * bpw
        pltpu.sync_copy(idx_hbm.at[pl.ds(base, bpw)], idx_v)
        pltpu.async_copy(table_hbm.at[idx_v], rows_v, sem).wait()
        pltpu.sync_copy(rows_v, out_hbm.at[pl.ds(base, bpw)])

    return gather_k


_sc_gather_cache = []


def _sc_gather(idx, table):
    if not _sc_gather_cache:
        _sc_gather_cache.append(_make_sc_gather())
    return _sc_gather_cache[0](idx, table)


def _finalize_body(x_ref, q_ref, quant_ref, loss_ref):
    x = x_ref[...]
    q = q_ref[...]
    dqx = q - x
    quant_ref[...] = x + dqx
    s = jnp.sum(dqx * dqx)
    loss_ref[...] = (1.25 * (s * (1.0 / (N * D)))).reshape(1, 1)


def _finalize(flat, q):
    return pl.pallas_call(
        _finalize_body,
        out_shape=[
            jax.ShapeDtypeStruct((N, D), jnp.float32),
            jax.ShapeDtypeStruct((1, 1), jnp.float32),
        ],
    )(flat, q)


def kernel(inputs, codebook):
    x = jnp.transpose(inputs, (0, 2, 3, 1))
    flat = x.reshape(-1, D)
    idx = _argmin_indices(flat, codebook).reshape(N)
    q = _sc_gather(idx, codebook)
    quant_flat, loss = _finalize(flat, q)
    quant = jnp.transpose(quant_flat.reshape(x.shape), (0, 3, 1, 2))
    return (quant, loss.reshape(()), idx)


# speedup vs baseline: 1.1765x; 1.0022x over previous
import functools

import jax
import jax.numpy as jnp
from jax import lax
from jax.experimental import pallas as pl
from jax.experimental.pallas import tpu as pltpu
from jax.experimental.pallas import tpu_sc as plsc

K = 8192
D = 256
N = 4096
B = 4
RB = N // B
CB = 8192


def _argmin_body(x_ref, c_ref, idx_ref, mn_ref, mi_ref):
    j = pl.program_id(1)
    nj = pl.num_programs(1)

    @pl.when(j == 0)
    def _():
        mn_ref[...] = jnp.full((RB, 1), jnp.inf, jnp.float32)
        mi_ref[...] = jnp.zeros((RB, 1), jnp.float32)

    x = x_ref[...]
    c = c_ref[...]
    xn = jnp.sum(x * x, axis=1, keepdims=True)
    cn = jnp.sum(c * c, axis=1)[None, :]
    mm2 = lax.dot_general(x * (-2.0), c, (((1,), (1,)), ((), ())),
                          preferred_element_type=jnp.float32)
    d = (xn + cn) + mm2
    m_loc = jnp.min(d, axis=1, keepdims=True)
    cols = lax.broadcasted_iota(jnp.int32, (1, CB), 1).astype(jnp.float32)
    i_loc = jnp.min(jnp.where(d == m_loc, cols, jnp.inf), axis=1, keepdims=True)
    better = m_loc < mn_ref[...]
    mi_ref[...] = jnp.where(better, i_loc + (j * CB).astype(jnp.float32),
                            mi_ref[...])
    mn_ref[...] = jnp.where(better, m_loc, mn_ref[...])

    @pl.when(j == nj - 1)
    def _():
        idx_ref[...] = mi_ref[...].astype(jnp.int32)


def _argmin_indices(flat, codebook):
    return pl.pallas_call(
        _argmin_body,
        grid=(B, K // CB),
        in_specs=[
            pl.BlockSpec((RB, D), lambda i, j: (i, 0)),
            pl.BlockSpec((CB, D), lambda i, j: (j, 0)),
        ],
        out_specs=pl.BlockSpec((RB, 1), lambda i, j: (i, 0)),
        out_shape=jax.ShapeDtypeStruct((N, 1), jnp.int32),
        scratch_shapes=[
            pltpu.VMEM((RB, 1), jnp.float32),
            pltpu.VMEM((RB, 1), jnp.float32),
        ],
        compiler_params=pltpu.CompilerParams(
            dimension_semantics=("parallel", "arbitrary")),
    )(flat, codebook)


def _make_sc_gather():
    info = plsc.get_sparse_core_info()
    nw = info.num_cores * info.num_subcores
    bpw = N // nw
    mesh = plsc.VectorSubcoreMesh(core_axis_name="c", subcore_axis_name="s")

    @functools.partial(
        pl.kernel,
        mesh=mesh,
        out_type=jax.ShapeDtypeStruct((N, D), jnp.float32),
        scratch_types=[
            pltpu.VMEM((bpw,), jnp.int32),
            pltpu.VMEM((bpw, D), jnp.float32),
            pltpu.SemaphoreType.DMA,
        ],
    )
    def gather_k(idx_hbm, table_hbm, out_hbm, idx_v, rows_v, sem):
        wid = lax.axis_index("s") * info.num_cores + lax.axis_index("c")
        base = wid * bpw
        pltpu.sync_copy(idx_hbm.at[pl.ds(base, bpw)], idx_v)
        pltpu.async_copy(table_hbm.at[idx_v], rows_v, sem).wait()
        pltpu.sync_copy(rows_v, out_hbm.at[pl.ds(base, bpw)])

    return gather_k


_sc_gather_cache = []


def _sc_gather(idx, table):
    if not _sc_gather_cache:
        _sc_gather_cache.append(_make_sc_gather())
    return _sc_gather_cache[0](idx, table)


def _finalize_body(x_ref, q_ref, quant_ref, loss_ref):
    x = x_ref[...]
    q = q_ref[...]
    dqx = q - x
    quant_ref[...] = x + dqx
    s = jnp.sum(dqx * dqx)
    loss_ref[...] = (1.25 * (s * (1.0 / (N * D)))).reshape(1, 1)


def _finalize(flat, q):
    return pl.pallas_call(
        _finalize_body,
        out_shape=[
            jax.ShapeDtypeStruct((N, D), jnp.float32),
            jax.ShapeDtypeStruct((1, 1), jnp.float32),
        ],
    )(flat, q)


def kernel(inputs, codebook):
    x = jnp.transpose(inputs, (0, 2, 3, 1))
    flat = x.reshape(-1, D)
    idx = _argmin_indices(flat, codebook).reshape(N)
    q = _sc_gather(idx, codebook)
    quant_flat, loss = _finalize(flat, q)
    quant = jnp.transpose(quant_flat.reshape(x.shape), (0, 3, 1, 2))
    return (quant, loss.reshape(()), idx)


# speedup vs baseline: 1.2943x; 1.1001x over previous
import functools

import jax
import jax.numpy as jnp
from jax import lax
from jax.experimental import pallas as pl
from jax.experimental.pallas import tpu as pltpu
from jax.experimental.pallas import tpu_sc as plsc

K = 8192
D = 256
N = 4096
B = 4
RB = N // B
CB = 8192


def _argmin_body(x_ref, c_ref, idx_ref, loss_ref, acc_ref):
    i = pl.program_id(0)
    x = x_ref[...]
    c = c_ref[...]
    xn = jnp.sum(x * x, axis=1, keepdims=True)
    cn = jnp.sum(c * c, axis=1)[None, :]
    mm2 = lax.dot_general(x * (-2.0), c, (((1,), (1,)), ((), ())),
                          preferred_element_type=jnp.float32)
    d = (xn + cn) + mm2
    m_loc = jnp.min(d, axis=1, keepdims=True)
    cols = lax.broadcasted_iota(jnp.int32, (1, CB), 1).astype(jnp.float32)
    i_loc = jnp.min(jnp.where(d == m_loc, cols, jnp.inf), axis=1, keepdims=True)
    idx_ref[...] = i_loc.astype(jnp.int32)
    s = jnp.sum(m_loc)

    @pl.when(i == 0)
    def _():
        acc_ref[0] = 0.0

    acc_ref[0] += s

    @pl.when(i == B - 1)
    def _():
        loss_ref[...] = (1.25 * (acc_ref[0] * (1.0 / (N * D)))).reshape(1, 1)


def _argmin_indices(flat, codebook):
    return pl.pallas_call(
        _argmin_body,
        grid=(B,),
        in_specs=[
            pl.BlockSpec((RB, D), lambda i: (i, 0)),
            pl.BlockSpec((CB, D), lambda i: (0, 0)),
        ],
        out_specs=[
            pl.BlockSpec((RB, 1), lambda i: (i, 0)),
            pl.BlockSpec((1, 1), lambda i: (0, 0)),
        ],
        out_shape=[
            jax.ShapeDtypeStruct((N, 1), jnp.int32),
            jax.ShapeDtypeStruct((1, 1), jnp.float32),
        ],
        scratch_shapes=[pltpu.SMEM((1,), jnp.float32)],
    )(flat, codebook)


def _make_sc_gather():
    info = plsc.get_sparse_core_info()
    nw = info.num_cores * info.num_subcores
    bpw = N // nw
    mesh = plsc.VectorSubcoreMesh(core_axis_name="c", subcore_axis_name="s")

    @functools.partial(
        pl.kernel,
        mesh=mesh,
        out_type=jax.ShapeDtypeStruct((N, D), jnp.float32),
        scratch_types=[
            pltpu.VMEM((bpw,), jnp.int32),
            pltpu.VMEM((bpw, D), jnp.float32),
            pltpu.SemaphoreType.DMA,
        ],
    )
    def gather_k(idx_hbm, table_hbm, out_hbm, idx_v, rows_v, sem):
        wid = lax.axis_index("s") * info.num_cores + lax.axis_index("c")
        base = wid * bpw
        pltpu.sync_copy(idx_hbm.at[pl.ds(base, bpw)], idx_v)
        pltpu.async_copy(table_hbm.at[idx_v], rows_v, sem).wait()
        pltpu.sync_copy(rows_v, out_hbm.at[pl.ds(base, bpw)])

    return gather_k


_sc_gather_cache = []


def _sc_gather(idx, table):
    if not _sc_gather_cache:
        _sc_gather_cache.append(_make_sc_gather())
    return _sc_gather_cache[0](idx, table)


def kernel(inputs, codebook):
    x = jnp.transpose(inputs, (0, 2, 3, 1))
    flat = x.reshape(-1, D)
    idx2, loss = _argmin_indices(flat, codebook)
    idx = idx2.reshape(N)
    q = _sc_gather(idx, codebook)
    quant = jnp.transpose(q.reshape(x.shape), (0, 3, 1, 2))
    return (quant, loss.reshape(()), idx)
